# baseline (speedup 1.0000x reference)
import functools

import jax
import jax.numpy as jnp
from jax import lax
from jax.experimental import pallas as pl
from jax.experimental.pallas import tpu as pltpu
from jax.experimental.pallas import tpu_sc as plsc

ROWS = 8192
DIM = 4096

BT = 512


def _transpose_body(x_ref, o_ref):
    o_ref[...] = x_ref[...].T


def _transpose(x, r, c):
    return pl.pallas_call(
        _transpose_body,
        grid=(c // BT, r // BT),
        in_specs=[pl.BlockSpec((BT, BT), lambda j, i: (i, j))],
        out_specs=pl.BlockSpec((BT, BT), lambda j, i: (j, i)),
        out_shape=jax.ShapeDtypeStruct((c, r), x.dtype),
    )(x)


NC = 2
NS = 16
NW = NC * NS
B_PER_W = DIM // NW
CH = 4
NCH = B_PER_W // CH


def _sc_gather_body(xt_hbm, idx_hbm, out_hbm, i_all, b0, b1, sg0, sg1):
    wid = lax.axis_index("s") * NC + lax.axis_index("c")
    row0 = wid * B_PER_W

    pltpu.sync_copy(idx_hbm.at[wid], i_all)

    pltpu.async_copy(xt_hbm.at[i_all.at[0]], b0, sg0)

    @pl.loop(0, NCH, step=2)
    def _(ci):
        pltpu.async_copy(xt_hbm.at[i_all.at[ci + 1]], b1, sg1)
        pltpu.make_async_copy(xt_hbm.at[i_all.at[ci]], b0, sg0).wait()
        pltpu.sync_copy(b0, out_hbm.at[pl.ds(row0 + ci * CH, CH)])

        @pl.when(ci + 2 < NCH)
        def _():
            pltpu.async_copy(xt_hbm.at[i_all.at[ci + 2]], b0, sg0)

        pltpu.make_async_copy(xt_hbm.at[i_all.at[ci + 1]], b1, sg1).wait()
        pltpu.sync_copy(b1, out_hbm.at[pl.ds(row0 + (ci + 1) * CH, CH)])


def _sc_gather(xt, idx3d):
    mesh = plsc.VectorSubcoreMesh(core_axis_name="c", subcore_axis_name="s")
    kfn = pl.kernel(
        _sc_gather_body,
        mesh=mesh,
        out_type=jax.ShapeDtypeStruct((DIM, ROWS), jnp.float32),
        scratch_types=[
            pltpu.VMEM((NCH, CH), jnp.int32),
            pltpu.VMEM((CH, ROWS), jnp.float32),
            pltpu.VMEM((CH, ROWS), jnp.float32),
            pltpu.SemaphoreType.DMA,
            pltpu.SemaphoreType.DMA,
        ],
    )
    return kfn(xt, idx3d)


def kernel(x, perm):
    idx3d = perm.reshape(NW, NCH, CH)
    xt = _transpose(x, ROWS, DIM)
    yt = _sc_gather(xt, idx3d)
    return _transpose(yt, DIM, ROWS)
